# baseline (speedup 1.0000x reference)
import functools

import jax
import jax.numpy as jnp
from jax import lax
from jax.experimental import pallas as pl
from jax.experimental.pallas import tpu as pltpu
from jax.experimental.pallas import tpu_sc as plsc

N_NODES = 100000
N_EDGES = 1600000

NP = 100096
ROWS = NP // 128
SL = NP // 16
NW = 32
GPW = 392
EGP = NW * GPW
EPAD = EGP * 128
CH = 56
NCH = GPW // CH

_mesh = functools.partial(
    plsc.VectorSubcoreMesh, core_axis_name="c", subcore_axis_name="s")


def _fill(ref, n, value):
    v = jnp.full((16,), value, jnp.float32)

    def body(j, _):
        ref[pl.ds(j * 16, 16)] = v
        return 0

    lax.fori_loop(0, n // 16, body, 0)


def _zero_shared(slice_v, shared, s):
    _fill(slice_v, SL, 0.0)
    pltpu.sync_copy(slice_v, shared.at[pl.ds(s * SL, SL)])


def _stage_shared(hbm, slice_v, shared, s):
    pltpu.sync_copy(hbm.at[pl.ds(s * SL, SL)], slice_v)
    pltpu.sync_copy(slice_v, shared.at[pl.ds(s * SL, SL)])


@functools.partial(
    pl.kernel,
    mesh=_mesh(),
    out_type=jax.ShapeDtypeStruct((2 * NP,), jnp.float32),
    scratch_types=[
        pltpu.VMEM((CH, 128), jnp.int32),
        pltpu.VMEM((128,), jnp.float32),
        pltpu.VMEM((SL,), jnp.float32),
        pltpu.VMEM((CH * 128,), jnp.float32),
        pltpu.VMEM_SHARED((NP,), jnp.float32),
        pltpu.SemaphoreType.DMA,
    ],
)
def _sc_degree(ei_hbm, out_hbm, idx_v, ones_v, slice_v, drain_v, acc_sh,
               sem):
    c = lax.axis_index("c")
    s = lax.axis_index("s")
    wid = s * 2 + c
    _fill(ones_v, 128, 1.0)
    _zero_shared(slice_v, acc_sh, s)
    plsc.subcore_barrier()
    base_g = wid * GPW

    def chunk(ci, _):
        pltpu.sync_copy(ei_hbm.at[pl.ds(EGP + base_g + ci * CH, CH), :],
                        idx_v)

        def fire(j, _):
            pltpu.async_copy(ones_v, acc_sh.at[idx_v.at[j]], sem, add=True)
            return 0

        lax.fori_loop(0, CH, fire, 0)
        pltpu.make_async_copy(
            out_hbm.at[pl.ds(0, CH * 128)], drain_v, sem).wait()
        return 0

    lax.fori_loop(0, NCH, chunk, 0)
    plsc.subcore_barrier()
    pltpu.sync_copy(acc_sh.at[pl.ds(s * SL, SL)], slice_v)
    pltpu.sync_copy(slice_v, out_hbm.at[pl.ds(c * NP + s * SL, SL)])


@functools.partial(
    pl.kernel,
    mesh=_mesh(),
    out_type=jax.ShapeDtypeStruct((2 * NP,), jnp.float32),
    scratch_types=[
        pltpu.VMEM((CH, 128), jnp.int32),
        pltpu.VMEM((CH, 128), jnp.int32),
        pltpu.VMEM((CH * 128,), jnp.float32),
        pltpu.VMEM((SL,), jnp.float32),
        pltpu.VMEM_SHARED((NP,), jnp.float32),
        pltpu.VMEM_SHARED((NP,), jnp.float32),
        pltpu.SemaphoreType.DMA,
        pltpu.SemaphoreType.DMA,
    ],
)
def _sc_segsum1(ei_hbm, y_hbm, out_hbm, src_v, dst_v, vals_v, slice_v,
                y_sh, acc_sh, sem_g, sem_s):
    c = lax.axis_index("c")
    s = lax.axis_index("s")
    wid = s * 2 + c
    _zero_shared(slice_v, acc_sh, s)
    _stage_shared(y_hbm, slice_v, y_sh, s)
    plsc.subcore_barrier()
    base_g = wid * GPW

    def chunk(ci, _):
        pltpu.sync_copy(ei_hbm.at[pl.ds(base_g + ci * CH, CH), :], src_v)
        pltpu.sync_copy(ei_hbm.at[pl.ds(EGP + base_g + ci * CH, CH), :],
                        dst_v)

        def fire_g(j, _):
            pltpu.async_copy(y_sh.at[src_v.at[j]], vals_v.at[pl.ds(j * 128, 128)], sem_g)
            return 0

        lax.fori_loop(0, CH, fire_g, 0)
        pltpu.make_async_copy(
            y_hbm.at[pl.ds(0, CH * 128)], vals_v, sem_g).wait()

        def fire_s(j, _):
            pltpu.async_copy(vals_v.at[pl.ds(j * 128, 128)], acc_sh.at[dst_v.at[j]], sem_s,
                             add=True)
            return 0

        lax.fori_loop(0, CH, fire_s, 0)
        pltpu.make_async_copy(
            y_hbm.at[pl.ds(0, CH * 128)], vals_v, sem_s).wait()
        return 0

    lax.fori_loop(0, NCH, chunk, 0)
    plsc.subcore_barrier()
    pltpu.sync_copy(acc_sh.at[pl.ds(s * SL, SL)], slice_v)
    pltpu.sync_copy(slice_v, out_hbm.at[pl.ds(c * NP + s * SL, SL)])


@functools.partial(
    pl.kernel,
    mesh=_mesh(),
    out_type=(jax.ShapeDtypeStruct((2 * NP,), jnp.float32),
              jax.ShapeDtypeStruct((2 * NP,), jnp.float32)),
    scratch_types=[
        pltpu.VMEM((CH, 128), jnp.int32),
        pltpu.VMEM((CH, 128), jnp.int32),
        pltpu.VMEM((CH * 128,), jnp.float32),
        pltpu.VMEM((CH * 128,), jnp.float32),
        pltpu.VMEM((SL,), jnp.float32),
        pltpu.VMEM_SHARED((NP,), jnp.float32),
        pltpu.VMEM_SHARED((NP,), jnp.float32),
        pltpu.VMEM_SHARED((NP,), jnp.float32),
        pltpu.VMEM_SHARED((NP,), jnp.float32),
        pltpu.SemaphoreType.DMA,
        pltpu.SemaphoreType.DMA,
    ],
)
def _sc_segsum2(ei_hbm, z0_hbm, z1_hbm, out0_hbm, out1_hbm, src_v, dst_v,
                vals0_v, vals1_v, slice_v, z0_sh, z1_sh, acc0_sh, acc1_sh,
                sem_g, sem_s):
    c = lax.axis_index("c")
    s = lax.axis_index("s")
    wid = s * 2 + c
    _zero_shared(slice_v, acc0_sh, s)
    _zero_shared(slice_v, acc1_sh, s)
    _stage_shared(z0_hbm, slice_v, z0_sh, s)
    _stage_shared(z1_hbm, slice_v, z1_sh, s)
    plsc.subcore_barrier()
    base_g = wid * GPW

    def chunk(ci, _):
        pltpu.sync_copy(ei_hbm.at[pl.ds(base_g + ci * CH, CH), :], src_v)
        pltpu.sync_copy(ei_hbm.at[pl.ds(EGP + base_g + ci * CH, CH), :],
                        dst_v)

        def fire_g(j, _):
            pltpu.async_copy(z0_sh.at[src_v.at[j]], vals0_v.at[pl.ds(j * 128, 128)], sem_g)
            pltpu.async_copy(z1_sh.at[src_v.at[j]], vals1_v.at[pl.ds(j * 128, 128)], sem_g)
            return 0

        lax.fori_loop(0, CH, fire_g, 0)
        pltpu.make_async_copy(
            z0_hbm.at[pl.ds(0, CH * 128)], vals0_v, sem_g).wait()
        pltpu.make_async_copy(
            z1_hbm.at[pl.ds(0, CH * 128)], vals1_v, sem_g).wait()

        def fire_s(j, _):
            pltpu.async_copy(vals0_v.at[pl.ds(j * 128, 128)], acc0_sh.at[dst_v.at[j]], sem_s,
                             add=True)
            pltpu.async_copy(vals1_v.at[pl.ds(j * 128, 128)], acc1_sh.at[dst_v.at[j]], sem_s,
                             add=True)
            return 0

        lax.fori_loop(0, CH, fire_s, 0)
        pltpu.make_async_copy(
            z0_hbm.at[pl.ds(0, CH * 128)], vals0_v, sem_s).wait()
        pltpu.make_async_copy(
            z1_hbm.at[pl.ds(0, CH * 128)], vals1_v, sem_s).wait()
        return 0

    lax.fori_loop(0, NCH, chunk, 0)
    plsc.subcore_barrier()
    pltpu.sync_copy(acc0_sh.at[pl.ds(s * SL, SL)], slice_v)
    pltpu.sync_copy(slice_v, out0_hbm.at[pl.ds(c * NP + s * SL, SL)])
    pltpu.sync_copy(acc1_sh.at[pl.ds(s * SL, SL)], slice_v)
    pltpu.sync_copy(slice_v, out1_hbm.at[pl.ds(c * NP + s * SL, SL)])


def _tc1_body(degp_ref, x_ref, dis_ref, y_ref):
    deg = degp_ref[0] + degp_ref[1] + 1.0
    dis = lax.rsqrt(deg)
    dis_ref[...] = dis
    y_ref[...] = x_ref[...] * dis


def _tc2_body(gp_ref, dis_ref, x_ref, w1_ref, b1_ref, w2_ref,
              h20_ref, h21_ref, z0_ref, z1_ref):
    dis = dis_ref[...]
    s1 = dis * (gp_ref[0] + gp_ref[1]) + dis * dis * x_ref[...]
    acc0 = jnp.zeros_like(s1)
    acc1 = jnp.zeros_like(s1)
    for j in range(16):
        t = jnp.maximum(s1 * w1_ref[0, j] + b1_ref[j], 0.0)
        acc0 += t * w2_ref[j, 0]
        acc1 += t * w2_ref[j, 1]
    h20_ref[...] = acc0
    h21_ref[...] = acc1
    z0_ref[...] = acc0 * dis
    z1_ref[...] = acc1 * dis


def _tc3_body(g0p_ref, g1p_ref, dis_ref, h20_ref, h21_ref, b2_ref,
              o0_ref, o1_ref):
    dis = dis_ref[...]
    d2 = dis * dis
    t0 = dis * (g0p_ref[0] + g0p_ref[1]) + d2 * h20_ref[...] + b2_ref[0]
    t1 = dis * (g1p_ref[0] + g1p_ref[1]) + d2 * h21_ref[...] + b2_ref[1]
    m = jnp.maximum(t0, t1)
    lse = jnp.log(jnp.exp(t0 - m) + jnp.exp(t1 - m))
    o0_ref[...] = t0 - m - lse
    o1_ref[...] = t1 - m - lse


_NODE = jax.ShapeDtypeStruct((ROWS, 128), jnp.float32)
_SSPEC = pl.BlockSpec(memory_space=pltpu.SMEM)


def _tc_call(body, n_in_vec, n_in_smem, n_out):
    return pl.pallas_call(
        body,
        out_shape=tuple(_NODE for _ in range(n_out)),
        in_specs=[pl.BlockSpec() for _ in range(n_in_vec)]
        + [_SSPEC for _ in range(n_in_smem)],
        out_specs=tuple(pl.BlockSpec() for _ in range(n_out)),
    )


def kernel(x, edge_index, W1, b1, W2, b2):
    ei = edge_index.astype(jnp.int32)
    pad = jnp.full((2, EPAD - N_EDGES), N_NODES, jnp.int32)
    ei3 = jnp.concatenate([ei, pad], axis=1).reshape(2 * EGP, 128)
    xf = jnp.pad(x[:, 0], (0, NP - N_NODES))

    degp = _sc_degree(ei3)

    dis, y = _tc_call(_tc1_body, 2, 0, 2)(
        degp.reshape(2, ROWS, 128), xf.reshape(ROWS, 128))

    gp = _sc_segsum1(ei3, y.reshape(NP))

    h20, h21, z0, z1 = _tc_call(_tc2_body, 3, 3, 4)(
        gp.reshape(2, ROWS, 128), dis, xf.reshape(ROWS, 128), W1, b1, W2)

    g0p, g1p = _sc_segsum2(ei3, z0.reshape(NP), z1.reshape(NP))

    o0, o1 = _tc_call(_tc3_body, 5, 1, 2)(
        g0p.reshape(2, ROWS, 128), g1p.reshape(2, ROWS, 128),
        dis, h20, h21, b2)

    return jnp.stack([o0.reshape(NP)[:N_NODES],
                      o1.reshape(NP)[:N_NODES]], axis=1)


# speedup vs baseline: 1.0904x; 1.0904x over previous
import functools

import jax
import jax.numpy as jnp
from jax import lax
from jax.experimental import pallas as pl
from jax.experimental.pallas import tpu as pltpu
from jax.experimental.pallas import tpu_sc as plsc

N_NODES = 100000
N_EDGES = 1600000

NP = 100096
ROWS = NP // 128
SL = NP // 16
NW = 32
GPW = 392
EGP = NW * GPW
EPAD = EGP * 128
CH = 56
NCH = GPW // CH
CHW = CH * 128
SUB = 8
NSUB = CH // SUB

_mesh = functools.partial(
    plsc.VectorSubcoreMesh, core_axis_name="c", subcore_axis_name="s")


def _fill(ref, n, value):
    v = jnp.full((16,), value, jnp.float32)

    def body(j, _):
        ref[pl.ds(j * 16, 16)] = v
        return 0

    lax.fori_loop(0, n // 16, body, 0)


def _zero_shared(stage_v, shared, s):
    _fill(stage_v, SL, 0.0)
    pltpu.sync_copy(stage_v.at[pl.ds(0, SL)], shared.at[pl.ds(s * SL, SL)])


def _write_partial(shared, stage_v, out_hbm, c, s):
    pltpu.sync_copy(shared.at[pl.ds(s * SL, SL)], stage_v.at[pl.ds(0, SL)])
    pltpu.sync_copy(stage_v.at[pl.ds(0, SL)],
                    out_hbm.at[pl.ds(c * NP + s * SL, SL)])


@functools.partial(
    pl.kernel,
    mesh=_mesh(),
    out_type=jax.ShapeDtypeStruct((2 * NP,), jnp.float32),
    scratch_types=[
        pltpu.VMEM((CH, 128), jnp.int32),
        pltpu.VMEM((128,), jnp.float32),
        pltpu.VMEM((CHW,), jnp.float32),
        pltpu.VMEM_SHARED((NP,), jnp.float32),
        pltpu.SemaphoreType.DMA,
    ],
)
def _sc_degree(ei2d_hbm, out_hbm, idx_v, ones_v, stage_v, acc_sh, sem):
    c = lax.axis_index("c")
    s = lax.axis_index("s")
    wid = s * 2 + c
    _fill(ones_v, 128, 1.0)
    _zero_shared(stage_v, acc_sh, s)
    plsc.subcore_barrier()
    base_g = wid * GPW

    def chunk(ci, _):
        pltpu.sync_copy(ei2d_hbm.at[pl.ds(EGP + base_g + ci * CH, CH), :],
                        idx_v)

        def fire(j, _):
            pltpu.async_copy(ones_v, acc_sh.at[idx_v.at[j]], sem, add=True)
            return 0

        lax.fori_loop(0, CH, fire, 0)
        pltpu.make_async_copy(
            out_hbm.at[pl.ds(0, CHW)], stage_v, sem).wait()
        return 0

    lax.fori_loop(0, NCH, chunk, 0)
    plsc.subcore_barrier()
    _write_partial(acc_sh, stage_v, out_hbm, c, s)


def _gather_sub(tbl_loc, src_f, vals_f, h):
    def body(k, _):
        base = h * (SUB * 128) + k * 64
        for u in range(4):
            idx = src_f[pl.ds(base + u * 16, 16)]
            vals_f[pl.ds(base + u * 16, 16)] = plsc.load_gather(
                tbl_loc, [idx])
        return 0

    lax.fori_loop(0, SUB * 2, body, 0)


@functools.partial(
    pl.kernel,
    mesh=_mesh(),
    out_type=jax.ShapeDtypeStruct((2 * NP,), jnp.float32),
    scratch_types=[
        pltpu.VMEM((CHW,), jnp.int32),
        pltpu.VMEM((CH, 128), jnp.int32),
        pltpu.VMEM((CHW,), jnp.float32),
        pltpu.VMEM((NP,), jnp.float32),
        pltpu.VMEM_SHARED((NP,), jnp.float32),
        pltpu.SemaphoreType.DMA,
    ],
    compiler_params=pltpu.CompilerParams(needs_layout_passes=False),
)
def _sc_segsum_local(ei2d_hbm, ei1d_hbm, tbl_hbm, out_hbm, src_f, dst_v,
                     vals_f, tbl_loc, acc_sh, sem_s):
    c = lax.axis_index("c")
    s = lax.axis_index("s")
    wid = s * 2 + c
    _zero_shared(vals_f, acc_sh, s)
    pltpu.sync_copy(tbl_hbm, tbl_loc)
    plsc.subcore_barrier()
    base_g = wid * GPW

    def chunk(ci, _):
        g0 = base_g + ci * CH
        pltpu.sync_copy(ei1d_hbm.at[pl.ds(g0 * 128, CHW)], src_f)
        pltpu.sync_copy(ei2d_hbm.at[pl.ds(EGP + g0, CH), :], dst_v)
        for h in range(NSUB):
            _gather_sub(tbl_loc, src_f, vals_f, h)
            for r in range(SUB):
                row = h * SUB + r
                pltpu.async_copy(vals_f.at[pl.ds(row * 128, 128)],
                                 acc_sh.at[dst_v.at[row]], sem_s, add=True)
        pltpu.make_async_copy(
            tbl_hbm.at[pl.ds(0, CHW)], vals_f, sem_s).wait()
        return 0

    lax.fori_loop(0, NCH, chunk, 0)
    plsc.subcore_barrier()
    _write_partial(acc_sh, vals_f, out_hbm, c, s)


def _unpack_sub(valsp_f, vals0_f, vals1_f, h):
    mask_hi = jnp.full((16,), -65536, jnp.int32)

    def body(k, _):
        base = h * (SUB * 128) + k * 64
        for u in range(4):
            w = valsp_f[pl.ds(base + u * 16, 16)]
            vals0_f[pl.ds(base + u * 16, 16)] = plsc.bitcast(
                lax.shift_left(w, 16), jnp.float32)
            vals1_f[pl.ds(base + u * 16, 16)] = plsc.bitcast(
                lax.bitwise_and(w, mask_hi), jnp.float32)
        return 0

    lax.fori_loop(0, SUB * 2, body, 0)


@functools.partial(
    pl.kernel,
    mesh=_mesh(),
    out_type=(jax.ShapeDtypeStruct((2 * NP,), jnp.float32),
              jax.ShapeDtypeStruct((2 * NP,), jnp.float32)),
    scratch_types=[
        pltpu.VMEM((CH, 128), jnp.int32),
        pltpu.VMEM((CH, 128), jnp.int32),
        pltpu.VMEM((CHW,), jnp.int32),
        pltpu.VMEM((CHW,), jnp.float32),
        pltpu.VMEM((CHW,), jnp.float32),
        pltpu.VMEM_SHARED((NP,), jnp.int32),
        pltpu.VMEM_SHARED((NP,), jnp.float32),
        pltpu.VMEM_SHARED((NP,), jnp.float32),
        pltpu.SemaphoreType.DMA,
        pltpu.SemaphoreType.DMA,
    ],
    compiler_params=pltpu.CompilerParams(needs_layout_passes=False),
)
def _sc_segsum_packed(ei2d_hbm, tbl_hbm, out0_hbm, out1_hbm,
                      src_v, dst_v, valsp_f, vals0_f, vals1_f,
                      tbl_sh, acc0_sh, acc1_sh, sem_g, sem_s):
    c = lax.axis_index("c")
    s = lax.axis_index("s")
    wid = s * 2 + c
    _zero_shared(vals0_f, acc0_sh, s)
    _zero_shared(vals0_f, acc1_sh, s)
    pltpu.sync_copy(tbl_hbm.at[pl.ds(s * SL, SL)],
                    valsp_f.at[pl.ds(0, SL)])
    pltpu.sync_copy(valsp_f.at[pl.ds(0, SL)], tbl_sh.at[pl.ds(s * SL, SL)])
    plsc.subcore_barrier()
    base_g = wid * GPW

    def chunk(ci, _):
        g0 = base_g + ci * CH
        pltpu.sync_copy(ei2d_hbm.at[pl.ds(g0, CH), :], src_v)
        pltpu.sync_copy(ei2d_hbm.at[pl.ds(EGP + g0, CH), :], dst_v)

        def fire_g(j, _):
            pltpu.async_copy(tbl_sh.at[src_v.at[j]],
                             valsp_f.at[pl.ds(j * 128, 128)], sem_g)
            return 0

        lax.fori_loop(0, CH, fire_g, 0)
        pltpu.make_async_copy(
            out0_hbm.at[pl.ds(0, CHW)], valsp_f, sem_g).wait()
        for h in range(NSUB):
            _unpack_sub(valsp_f, vals0_f, vals1_f, h)
            for r in range(SUB):
                row = h * SUB + r
                pltpu.async_copy(vals0_f.at[pl.ds(row * 128, 128)],
                                 acc0_sh.at[dst_v.at[row]], sem_s, add=True)
                pltpu.async_copy(vals1_f.at[pl.ds(row * 128, 128)],
                                 acc1_sh.at[dst_v.at[row]], sem_s, add=True)
        pltpu.make_async_copy(
            out0_hbm.at[pl.ds(0, CHW)], vals0_f, sem_s).wait()
        pltpu.make_async_copy(
            out0_hbm.at[pl.ds(0, CHW)], vals1_f, sem_s).wait()
        return 0

    lax.fori_loop(0, NCH, chunk, 0)
    plsc.subcore_barrier()
    _write_partial(acc0_sh, vals0_f, out0_hbm, c, s)
    _write_partial(acc1_sh, vals1_f, out1_hbm, c, s)


def _round_bf16_bits(z):
    i = lax.bitcast_convert_type(z, jnp.int32)
    odd = lax.bitwise_and(lax.shift_right_logical(i, 16), 1)
    r = lax.shift_right_logical(i + 0x7FFF + odd, 16)
    return lax.bitwise_and(r, 0xFFFF)


def _tc1_body(degp_ref, x_ref, dis_ref, y_ref):
    deg = degp_ref[0] + degp_ref[1] + 1.0
    dis = lax.rsqrt(deg)
    dis_ref[...] = dis
    y_ref[...] = x_ref[...] * dis


def _tc2_body(gp_ref, dis_ref, x_ref, w1_ref, b1_ref, w2_ref,
              h20_ref, h21_ref, zp_ref):
    dis = dis_ref[...]
    s1 = dis * (gp_ref[0] + gp_ref[1]) + dis * dis * x_ref[...]
    acc0 = jnp.zeros_like(s1)
    acc1 = jnp.zeros_like(s1)
    for j in range(16):
        t = jnp.maximum(s1 * w1_ref[0, j] + b1_ref[j], 0.0)
        acc0 += t * w2_ref[j, 0]
        acc1 += t * w2_ref[j, 1]
    h20_ref[...] = acc0
    h21_ref[...] = acc1
    b0 = _round_bf16_bits(acc0 * dis)
    b1b = _round_bf16_bits(acc1 * dis)
    zp_ref[...] = lax.bitwise_or(lax.shift_left(b1b, 16), b0)


def _tc3_body(g0p_ref, g1p_ref, dis_ref, h20_ref, h21_ref, b2_ref,
              o0_ref, o1_ref):
    dis = dis_ref[...]
    d2 = dis * dis
    t0 = dis * (g0p_ref[0] + g0p_ref[1]) + d2 * h20_ref[...] + b2_ref[0]
    t1 = dis * (g1p_ref[0] + g1p_ref[1]) + d2 * h21_ref[...] + b2_ref[1]
    m = jnp.maximum(t0, t1)
    lse = jnp.log(jnp.exp(t0 - m) + jnp.exp(t1 - m))
    o0_ref[...] = t0 - m - lse
    o1_ref[...] = t1 - m - lse


_NODE_F = jax.ShapeDtypeStruct((ROWS, 128), jnp.float32)
_NODE_I = jax.ShapeDtypeStruct((ROWS, 128), jnp.int32)
_SSPEC = pl.BlockSpec(memory_space=pltpu.SMEM)


def _tc_call(body, n_in_vec, n_in_smem, out_shapes):
    return pl.pallas_call(
        body,
        out_shape=out_shapes,
        in_specs=[pl.BlockSpec() for _ in range(n_in_vec)]
        + [_SSPEC for _ in range(n_in_smem)],
        out_specs=tuple(pl.BlockSpec() for _ in out_shapes),
    )


def kernel(x, edge_index, W1, b1, W2, b2):
    ei = edge_index.astype(jnp.int32)
    pad = jnp.full((2, EPAD - N_EDGES), N_NODES, jnp.int32)
    eic = jnp.concatenate([ei, pad], axis=1)
    ei2d = eic.reshape(2 * EGP, 128)
    ei1d = eic.reshape(2 * EPAD)
    xf = jnp.pad(x[:, 0], (0, NP - N_NODES))

    degp = _sc_degree(ei2d)

    dis, y = _tc_call(_tc1_body, 2, 0, (_NODE_F, _NODE_F))(
        degp.reshape(2, ROWS, 128), xf.reshape(ROWS, 128))

    gp = _sc_segsum_local(ei2d, ei1d, y.reshape(NP))

    h20, h21, zp = _tc_call(_tc2_body, 3, 3, (_NODE_F, _NODE_F, _NODE_I))(
        gp.reshape(2, ROWS, 128), dis, xf.reshape(ROWS, 128), W1, b1, W2)

    g0p, g1p = _sc_segsum_packed(ei2d, zp.reshape(NP))

    o0, o1 = _tc_call(_tc3_body, 5, 1, (_NODE_F, _NODE_F))(
        g0p.reshape(2, ROWS, 128), g1p.reshape(2, ROWS, 128),
        dis, h20, h21, b2)

    return jnp.stack([o0.reshape(NP)[:N_NODES],
                      o1.reshape(NP)[:N_NODES]], axis=1)
